# baseline (device time: 68809 ns/iter reference)
import jax
import jax.numpy as jnp
from jax import lax
from jax.experimental import pallas as pl
from jax.experimental.pallas import tpu as pltpu

N_DEV = 8

BUTTERFLIES = (
    (0, 1408, (1, 3, 4)),
    (1408, 1408, (3, 4, 1)),
    (2816, 1280, (4, 1, 3)),
)

_COMM_OFFS = {}
_off = 0
for _b, (_base, _rows, _order) in enumerate(BUTTERFLIES):
    _r = _rows
    for _s in range(3):
        _r //= 2
        _COMM_OFFS[(_b, _s)] = _off
        _off += _r
COMM_ROWS = _off

N_SEMS = 13


def _m8(v):
    return pl.multiple_of(v, 8)


def kernel(x):
    m, n = x.shape
    assert m == sum(rows for _, rows, _ in BUTTERFLIES)

    def body(
        x_hbm, out_hbm, gat_ref, xv_ref, comm_ref,
        in_sems, out_sems, send_sems, recv_sems,
    ):
        p = lax.axis_index("i")
        b0 = jnp.bitwise_and(p, 1)
        b1 = jnp.bitwise_and(p // 2, 1)
        b2 = jnp.bitwise_and(p // 4, 1)
        keep_fns = {
            1: jnp.bitwise_xor(b0, b1) == 0,
            3: b1 == 0,
            4: b2 == 0,
        }

        def rs_parts(s, order, lo, length):
            half = length // 2
            keep_lo = keep_fns[order[s]]
            send_off = _m8(jnp.where(keep_lo, lo + half, lo))
            keep_off = _m8(jnp.where(keep_lo, lo, lo + half))
            quarter = half // 2
            if s < 2:
                nk = keep_fns[order[s + 1]]
                crit_rel = jnp.where(nk, quarter, 0)
            else:
                crit_rel = jnp.int32(0)
            rest_rel = quarter - crit_rel
            return send_off, keep_off, half, quarter, crit_rel, rest_rel

        def start_rs_sub(s, b, order, send_off, quarter, rel, sub):
            qdev = jnp.bitwise_xor(p, order[s])
            rdma = pltpu.make_async_remote_copy(
                src_ref=gat_ref.at[pl.ds(_m8(send_off + rel), quarter), :],
                dst_ref=comm_ref.at[
                    pl.ds(_m8(_COMM_OFFS[(b, s)] + rel), quarter), :
                ],
                send_sem=send_sems.at[2 * s + sub, b],
                recv_sem=recv_sems.at[2 * s + sub, b],
                device_id=(qdev,),
                device_id_type=pl.DeviceIdType.MESH,
            )
            rdma.start()
            return rdma

        def start_rs_sends(s, b, order, send_off, quarter, crit_rel, rest_rel):
            return [
                start_rs_sub(s, b, order, send_off, quarter, crit_rel, 0),
                start_rs_sub(s, b, order, send_off, quarter, rest_rel, 1),
            ]

        def start_ag_push(sem_idx, b, qdev, lo, length):
            rdma = pltpu.make_async_remote_copy(
                src_ref=gat_ref.at[pl.ds(_m8(lo), length), :],
                dst_ref=gat_ref.at[pl.ds(_m8(lo), length), :],
                send_sem=send_sems.at[sem_idx, b],
                recv_sem=recv_sems.at[sem_idx, b],
                device_id=(qdev,),
                device_id_type=pl.DeviceIdType.MESH,
            )
            rdma.start()
            return rdma

        def start_out_dma(b, slot, lo, length):
            cp = pltpu.make_async_copy(
                gat_ref.at[pl.ds(_m8(lo), length), :],
                out_hbm.at[pl.ds(_m8(lo), length), :],
                out_sems.at[b, slot],
            )
            cp.start()
            return cp

        def add_block(dst_off, rows_, src_off):
            gat_ref[pl.ds(_m8(dst_off), rows_), :] = (
                gat_ref[pl.ds(_m8(dst_off), rows_), :]
                + comm_ref[pl.ds(_m8(src_off), rows_), :]
            )

        in_dmas = []
        for b, (base, rows, order) in enumerate(BUTTERFLIES):
            half = rows // 2
            dmas = []
            for h in range(2):
                cp = pltpu.make_async_copy(
                    x_hbm.at[pl.ds(base + h * half, half), :],
                    xv_ref.at[pl.ds(base + h * half, half), :],
                    in_sems.at[b, h],
                )
                cp.start()
                dmas.append(cp)
            in_dmas.append(dmas)

        barrier = pltpu.get_barrier_semaphore()
        for mask in (1, 3, 4):
            q = jnp.bitwise_xor(p, mask)
            pl.semaphore_signal(
                barrier, inc=1, device_id=(q,),
                device_id_type=pl.DeviceIdType.MESH,
            )
        pl.semaphore_wait(barrier, 3)

        rdmas = {}
        out_dmas = []
        states = []
        keep_pend = []
        for b, (base, rows, order) in enumerate(BUTTERFLIES):
            in_dmas[b][0].wait()
            in_dmas[b][1].wait()
            send_off, keep_off, half, quarter, crit_rel, rest_rel = rs_parts(
                0, order, jnp.int32(base), rows
            )
            gat_ref[pl.ds(_m8(send_off + crit_rel), quarter), :] = (
                xv_ref[pl.ds(_m8(send_off + crit_rel), quarter), :]
                .astype(gat_ref.dtype)
            )
            crit = start_rs_sub(0, b, order, send_off, quarter, crit_rel, 0)
            gat_ref[pl.ds(_m8(send_off + rest_rel), quarter), :] = (
                xv_ref[pl.ds(_m8(send_off + rest_rel), quarter), :]
                .astype(gat_ref.dtype)
            )
            rest = start_rs_sub(0, b, order, send_off, quarter, rest_rel, 1)
            rdmas[(0, b)] = [crit, rest]
            states.append((jnp.int32(base), rows))
            keep_pend.append((keep_off, half))
        for keep_off, half in keep_pend:
            gat_ref[pl.ds(keep_off, half), :] = (
                xv_ref[pl.ds(keep_off, half), :].astype(gat_ref.dtype)
            )

        ag_meta = []
        for s in range(3):
            pend = []
            for b, (base, rows, order) in enumerate(BUTTERFLIES):
                lo, length = states[b]
                send_off, keep_off, half, quarter, crit_rel, rest_rel = (
                    rs_parts(s, order, lo, length)
                )
                crit_rdma, rest_rdma = rdmas[(s, b)]
                crit_rdma.wait_recv()
                add_block(
                    keep_off + crit_rel, quarter, _COMM_OFFS[(b, s)] + crit_rel
                )
                states[b] = (keep_off, half)
                if s < 2:
                    nso, _, _, nq, ncr, nrr = rs_parts(s + 1, order, keep_off, half)
                    rdmas[(s + 1, b)] = start_rs_sends(
                        s + 1, b, order, nso, nq, ncr, nrr
                    )
                pend.append((rest_rdma, keep_off, half, quarter, rest_rel))
            for b, (rest_rdma, keep_off, half, quarter, rest_rel) in enumerate(pend):
                order = BUTTERFLIES[b][2]
                rest_rdma.wait_recv()
                add_block(
                    keep_off + rest_rel, quarter, _COMM_OFFS[(b, s)] + rest_rel
                )
                if s == 2:
                    L = half
                    lo_f = keep_off
                    rows_b = BUTTERFLIES[b][1]
                    q0 = jnp.bitwise_xor(p, order[2])
                    q1 = jnp.bitwise_xor(p, order[1])
                    q2 = jnp.bitwise_xor(p, order[0])
                    k0 = keep_fns[order[2]]
                    k1 = keep_fns[order[1]]
                    k2 = keep_fns[order[0]]
                    r0 = _m8(jnp.where(k0, lo_f + L, lo_f - L))
                    lo1 = _m8(jnp.where(k0, lo_f, lo_f - L))
                    r1 = _m8(jnp.where(k1, lo1 + 2 * L, lo1 - 2 * L))
                    lo2 = _m8(jnp.where(k1, lo1, lo1 - 2 * L))
                    r2 = _m8(jnp.where(k2, lo2 + 4 * L, lo2 - 4 * L))
                    q1_lo_f = _m8(
                        lo_f + jnp.where(k1, rows_b // 4, -(rows_b // 4))
                    )
                    q1_r0 = _m8(jnp.where(k0, q1_lo_f + L, q1_lo_f - L))
                    rdmas[("ag_p0_q0", b)] = start_ag_push(6, b, q0, lo_f, L)
                    rdmas[("ag_p0_q1", b)] = start_ag_push(7, b, q1, lo_f, L)
                    rdmas[("ag_p0_q2", b)] = start_ag_push(9, b, q2, lo_f, L)
                    out_dmas.append(start_out_dma(b, 0, lo_f, L))
                    ag_meta.append((q0, q1, q2, r0, q1_lo_f, q1_r0, r2, L))

        for b, (q0, q1, q2, r0, q1_lo_f, q1_r0, r2, L) in enumerate(ag_meta):
            rdmas[("ag_p0_q0", b)].wait_recv()
            rdmas[("ag_r0_q1", b)] = start_ag_push(8, b, q1, r0, L)
            rdmas[("ag_r0_q2", b)] = start_ag_push(10, b, q2, r0, L)
            out_dmas.append(start_out_dma(b, 1, r0, L))
        for b, (q0, q1, q2, r0, q1_lo_f, q1_r0, r2, L) in enumerate(ag_meta):
            rdmas[("ag_p0_q1", b)].wait_recv()
            rdmas[("ag_q1p0_q2", b)] = start_ag_push(11, b, q2, q1_lo_f, L)
            out_dmas.append(start_out_dma(b, 2, q1_lo_f, L))
        for b, (q0, q1, q2, r0, q1_lo_f, q1_r0, r2, L) in enumerate(ag_meta):
            rdmas[("ag_r0_q1", b)].wait_recv()
            rdmas[("ag_q1r0_q2", b)] = start_ag_push(12, b, q2, q1_r0, L)
            out_dmas.append(start_out_dma(b, 3, q1_r0, L))
        for b, (q0, q1, q2, r0, q1_lo_f, q1_r0, r2, L) in enumerate(ag_meta):
            rdmas[("ag_p0_q2", b)].wait_recv()
            rdmas[("ag_r0_q2", b)].wait_recv()
            rdmas[("ag_q1p0_q2", b)].wait_recv()
            rdmas[("ag_q1r0_q2", b)].wait_recv()
            out_dmas.append(start_out_dma(b, 4, r2, 4 * L))

        for v in rdmas.values():
            for rdma in v if isinstance(v, list) else [v]:
                rdma.wait_send()
        for cp in out_dmas:
            cp.wait()

    return pl.pallas_call(
        body,
        out_shape=jax.ShapeDtypeStruct((m, n), jnp.bfloat16),
        in_specs=[pl.BlockSpec(memory_space=pl.ANY)],
        out_specs=pl.BlockSpec(memory_space=pl.ANY),
        scratch_shapes=[
            pltpu.VMEM((m, n), jnp.bfloat16),
            pltpu.VMEM((m, n), x.dtype),
            pltpu.VMEM((COMM_ROWS, n), jnp.bfloat16),
            pltpu.SemaphoreType.DMA((len(BUTTERFLIES), 2)),
            pltpu.SemaphoreType.DMA((len(BUTTERFLIES), 5)),
            pltpu.SemaphoreType.DMA((N_SEMS, len(BUTTERFLIES))),
            pltpu.SemaphoreType.DMA((N_SEMS, len(BUTTERFLIES))),
        ],
        compiler_params=pltpu.CompilerParams(collective_id=0),
    )(x)


# device time: 68353 ns/iter; 1.0067x vs baseline; 1.0067x over previous
import jax
import jax.numpy as jnp
from jax import lax
from jax.experimental import pallas as pl
from jax.experimental.pallas import tpu as pltpu

N_DEV = 8

BUTTERFLIES = (
    (0, 1408, (1, 3, 4)),
    (1408, 1408, (3, 4, 1)),
    (2816, 1280, (4, 1, 3)),
)

_COMM_OFFS = {}
_off = 0
for _b, (_base, _rows, _order) in enumerate(BUTTERFLIES):
    _r = _rows
    for _s in range(3):
        _r //= 2
        _COMM_OFFS[(_b, _s)] = _off
        _off += _r
COMM_ROWS = _off

N_SEMS = 13


def _m8(v):
    return pl.multiple_of(v, 8)


def kernel(x):
    m, n = x.shape
    assert m == sum(rows for _, rows, _ in BUTTERFLIES)

    def body(x_hbm, out_ref, xv_ref, comm_ref, in_sems, send_sems, recv_sems):
        p = lax.axis_index("i")
        b0 = jnp.bitwise_and(p, 1)
        b1 = jnp.bitwise_and(p // 2, 1)
        b2 = jnp.bitwise_and(p // 4, 1)
        keep_fns = {
            1: jnp.bitwise_xor(b0, b1) == 0,
            3: b1 == 0,
            4: b2 == 0,
        }

        def rs_parts(s, order, lo, length):
            half = length // 2
            keep_lo = keep_fns[order[s]]
            send_off = _m8(jnp.where(keep_lo, lo + half, lo))
            keep_off = _m8(jnp.where(keep_lo, lo, lo + half))
            quarter = half // 2
            if s < 2:
                nk = keep_fns[order[s + 1]]
                crit_rel = jnp.where(nk, quarter, 0)
            else:
                crit_rel = jnp.int32(0)
            rest_rel = quarter - crit_rel
            return send_off, keep_off, half, quarter, crit_rel, rest_rel

        def start_rs_sub(s, b, order, send_off, quarter, rel, sub):
            qdev = jnp.bitwise_xor(p, order[s])
            rdma = pltpu.make_async_remote_copy(
                src_ref=out_ref.at[pl.ds(_m8(send_off + rel), quarter), :],
                dst_ref=comm_ref.at[
                    pl.ds(_m8(_COMM_OFFS[(b, s)] + rel), quarter), :
                ],
                send_sem=send_sems.at[2 * s + sub, b],
                recv_sem=recv_sems.at[2 * s + sub, b],
                device_id=(qdev,),
                device_id_type=pl.DeviceIdType.MESH,
            )
            rdma.start()
            return rdma

        def start_rs_sends(s, b, order, send_off, quarter, crit_rel, rest_rel):
            return [
                start_rs_sub(s, b, order, send_off, quarter, crit_rel, 0),
                start_rs_sub(s, b, order, send_off, quarter, rest_rel, 1),
            ]

        def start_ag_push(sem_idx, b, qdev, lo, length):
            rdma = pltpu.make_async_remote_copy(
                src_ref=out_ref.at[pl.ds(_m8(lo), length), :],
                dst_ref=out_ref.at[pl.ds(_m8(lo), length), :],
                send_sem=send_sems.at[sem_idx, b],
                recv_sem=recv_sems.at[sem_idx, b],
                device_id=(qdev,),
                device_id_type=pl.DeviceIdType.MESH,
            )
            rdma.start()
            return rdma

        def add_block(dst_off, rows_, src_off):
            out_ref[pl.ds(_m8(dst_off), rows_), :] = (
                out_ref[pl.ds(_m8(dst_off), rows_), :]
                + comm_ref[pl.ds(_m8(src_off), rows_), :]
            )

        in_dmas = []
        for b, (base, rows, order) in enumerate(BUTTERFLIES):
            half = rows // 2
            dmas = []
            for h in range(2):
                cp = pltpu.make_async_copy(
                    x_hbm.at[pl.ds(base + h * half, half), :],
                    xv_ref.at[pl.ds(base + h * half, half), :],
                    in_sems.at[b, h],
                )
                cp.start()
                dmas.append(cp)
            in_dmas.append(dmas)

        barrier = pltpu.get_barrier_semaphore()
        for mask in (1, 3, 4):
            q = jnp.bitwise_xor(p, mask)
            pl.semaphore_signal(
                barrier, inc=1, device_id=(q,),
                device_id_type=pl.DeviceIdType.MESH,
            )
        pl.semaphore_wait(barrier, 3)

        rdmas = {}
        states = []
        keep_pend = []
        for b, (base, rows, order) in enumerate(BUTTERFLIES):
            in_dmas[b][0].wait()
            in_dmas[b][1].wait()
            send_off, keep_off, half, quarter, crit_rel, rest_rel = rs_parts(
                0, order, jnp.int32(base), rows
            )
            out_ref[pl.ds(_m8(send_off + crit_rel), quarter), :] = (
                xv_ref[pl.ds(_m8(send_off + crit_rel), quarter), :]
                .astype(out_ref.dtype)
            )
            crit = start_rs_sub(0, b, order, send_off, quarter, crit_rel, 0)
            out_ref[pl.ds(_m8(send_off + rest_rel), quarter), :] = (
                xv_ref[pl.ds(_m8(send_off + rest_rel), quarter), :]
                .astype(out_ref.dtype)
            )
            rest = start_rs_sub(0, b, order, send_off, quarter, rest_rel, 1)
            rdmas[(0, b)] = [crit, rest]
            states.append((jnp.int32(base), rows))
            keep_pend.append((keep_off, half))
        for keep_off, half in keep_pend:
            out_ref[pl.ds(keep_off, half), :] = (
                xv_ref[pl.ds(keep_off, half), :].astype(out_ref.dtype)
            )

        ag_meta = []
        for s in range(3):
            pend = []
            for b, (base, rows, order) in enumerate(BUTTERFLIES):
                lo, length = states[b]
                send_off, keep_off, half, quarter, crit_rel, rest_rel = (
                    rs_parts(s, order, lo, length)
                )
                crit_rdma, rest_rdma = rdmas[(s, b)]
                crit_rdma.wait_recv()
                add_block(
                    keep_off + crit_rel, quarter, _COMM_OFFS[(b, s)] + crit_rel
                )
                states[b] = (keep_off, half)
                if s < 2:
                    nso, _, _, nq, ncr, nrr = rs_parts(s + 1, order, keep_off, half)
                    rdmas[(s + 1, b)] = start_rs_sends(
                        s + 1, b, order, nso, nq, ncr, nrr
                    )
                pend.append((rest_rdma, keep_off, half, quarter, rest_rel))
            for b, (rest_rdma, keep_off, half, quarter, rest_rel) in enumerate(pend):
                order = BUTTERFLIES[b][2]
                rest_rdma.wait_recv()
                add_block(
                    keep_off + rest_rel, quarter, _COMM_OFFS[(b, s)] + rest_rel
                )
                if s == 2:
                    L = half
                    lo_f = keep_off
                    rows_b = BUTTERFLIES[b][1]
                    q0 = jnp.bitwise_xor(p, order[2])
                    q1 = jnp.bitwise_xor(p, order[1])
                    q2 = jnp.bitwise_xor(p, order[0])
                    k0 = keep_fns[order[2]]
                    k1 = keep_fns[order[1]]
                    r0 = _m8(jnp.where(k0, lo_f + L, lo_f - L))
                    q1_lo_f = _m8(
                        lo_f + jnp.where(k1, rows_b // 4, -(rows_b // 4))
                    )
                    q1_r0 = _m8(jnp.where(k0, q1_lo_f + L, q1_lo_f - L))
                    rdmas[("ag_p0_q0", b)] = start_ag_push(6, b, q0, lo_f, L)
                    rdmas[("ag_p0_q1", b)] = start_ag_push(7, b, q1, lo_f, L)
                    rdmas[("ag_p0_q2", b)] = start_ag_push(9, b, q2, lo_f, L)
                    ag_meta.append((q0, q1, q2, r0, q1_lo_f, q1_r0, L))

        for b, (q0, q1, q2, r0, q1_lo_f, q1_r0, L) in enumerate(ag_meta):
            rdmas[("ag_p0_q0", b)].wait_recv()
            rdmas[("ag_r0_q1", b)] = start_ag_push(8, b, q1, r0, L)
            rdmas[("ag_r0_q2", b)] = start_ag_push(10, b, q2, r0, L)
        for b, (q0, q1, q2, r0, q1_lo_f, q1_r0, L) in enumerate(ag_meta):
            rdmas[("ag_p0_q1", b)].wait_recv()
            rdmas[("ag_q1p0_q2", b)] = start_ag_push(11, b, q2, q1_lo_f, L)
        for b, (q0, q1, q2, r0, q1_lo_f, q1_r0, L) in enumerate(ag_meta):
            rdmas[("ag_r0_q1", b)].wait_recv()
            rdmas[("ag_q1r0_q2", b)] = start_ag_push(12, b, q2, q1_r0, L)
        for b, (q0, q1, q2, r0, q1_lo_f, q1_r0, L) in enumerate(ag_meta):
            rdmas[("ag_p0_q2", b)].wait_recv()
            rdmas[("ag_r0_q2", b)].wait_recv()
            rdmas[("ag_q1p0_q2", b)].wait_recv()
            rdmas[("ag_q1r0_q2", b)].wait_recv()

        for v in rdmas.values():
            for rdma in v if isinstance(v, list) else [v]:
                rdma.wait_send()

    return pl.pallas_call(
        body,
        out_shape=jax.ShapeDtypeStruct((m, n), jnp.bfloat16),
        in_specs=[pl.BlockSpec(memory_space=pl.ANY)],
        out_specs=pl.BlockSpec(memory_space=pltpu.VMEM),
        scratch_shapes=[
            pltpu.VMEM((m, n), x.dtype),
            pltpu.VMEM((COMM_ROWS, n), jnp.bfloat16),
            pltpu.SemaphoreType.DMA((len(BUTTERFLIES), 2)),
            pltpu.SemaphoreType.DMA((N_SEMS, len(BUTTERFLIES))),
            pltpu.SemaphoreType.DMA((N_SEMS, len(BUTTERFLIES))),
        ],
        compiler_params=pltpu.CompilerParams(collective_id=0),
    )(x)
